# Initial kernel scaffold; baseline (speedup 1.0000x reference)
import jax
import jax.numpy as jnp
from jax.experimental import pallas as pl


def kernel(x, edge_index, drone_feat, batch, node_w, node_b, drone_w, drone_b, edge_w1, edge_b1, edge_w2, edge_b2, self_w, self_b, out_w, out_b, ln_g, ln_b, proj_w, proj_b):
    raise NotImplementedError("write your pallas kernel here")


# speedup vs baseline: 4.6763x; 4.6763x over previous
import functools

import jax
import jax.numpy as jnp
from jax import lax
from jax.experimental import pallas as pl
from jax.experimental.pallas import tpu as pltpu
from jax.experimental.pallas import tpu_sc as plsc

N_NODES = 10000
N_EDGES = 320000
HID = 128
N_OUT = 64
N_BATCH = 64
D_DRONE = 51
N_LAYERS = 3
LANES = 16

NC = 2
NS = 16
NW = NC * NS
EPW = N_EDGES // NW
CHUNK = 80
NCHUNK = EPW // CHUNK
CPR = 400
NCP = N_NODES // CPR

BLK = 400
GRID = N_NODES // BLK
F32 = jnp.float32

_SC_MESH = plsc.VectorSubcoreMesh(core_axis_name="c", subcore_axis_name="s")


def _edge_chunk_loop(wid, a_hbm, c_hbm, src_hbm, dst_hbm,
                     src_v, dst_v, a_v, c_v, s_sh, sem1, sem2,
                     ones_v=None, cnt_sh=None):
    def chunk_body(j, carry):
        off = wid * EPW + j * CHUNK
        pltpu.sync_copy(src_hbm.at[pl.ds(off, CHUNK)], src_v)
        pltpu.sync_copy(dst_hbm.at[pl.ds(off, CHUNK)], dst_v)
        cp1 = pltpu.async_copy(a_hbm.at[src_v], a_v, sem1)
        cp2 = pltpu.async_copy(c_hbm.at[dst_v], c_v, sem2)
        cp1.wait()
        cp2.wait()

        def row_body(i, cc):
            for g in range(HID // LANES):
                sl = pl.ds(g * LANES, LANES)
                a_v[i, sl] = jnp.maximum(a_v[i, sl] - c_v[i, sl], 0.0)
            return cc

        lax.fori_loop(0, CHUNK, row_body, 0)
        pltpu.sync_copy(a_v, s_sh.at[dst_v], add=True)
        if cnt_sh is not None:
            pltpu.sync_copy(ones_v, cnt_sh.at[dst_v], add=True)
        return carry

    lax.fori_loop(0, NCHUNK, chunk_body, 0)


def _rows_copy(sid, pairs):
    r0 = sid * CPR
    for src, dst in pairs:
        pltpu.sync_copy(src(r0), dst(r0))

    @pl.when(sid + NS < NCP)
    def _():
        r1 = (sid + NS) * CPR
        for src, dst in pairs:
            pltpu.sync_copy(src(r1), dst(r1))


@functools.partial(
    pl.kernel,
    out_type=(
        jax.ShapeDtypeStruct((NC, N_NODES, HID), F32),
        jax.ShapeDtypeStruct((NC, N_NODES, HID), F32),
    ),
    mesh=_SC_MESH,
    scratch_types=[
        pltpu.VMEM((CHUNK,), jnp.int32),
        pltpu.VMEM((CHUNK,), jnp.int32),
        pltpu.VMEM((CHUNK, HID), F32),
        pltpu.VMEM((CHUNK, HID), F32),
        pltpu.VMEM_SHARED((N_NODES, HID), F32),
        pltpu.SemaphoreType.DMA,
        pltpu.SemaphoreType.DMA,
    ],
)
def _sc_edge_cnt(a_hbm, c_hbm, src_hbm, dst_hbm, zs_hbm,
                 s_out, cnt_out,
                 src_v, dst_v, a_v, c_v, s_sh, sem1, sem2):
    cid = lax.axis_index("c")
    sid = lax.axis_index("s")
    wid = sid * NC + cid
    _rows_copy(sid, [
        (lambda r: zs_hbm.at[pl.ds(r, CPR)], lambda r: s_sh.at[pl.ds(r, CPR)]),
    ])
    plsc.subcore_barrier()
    _edge_chunk_loop(wid, a_hbm, c_hbm, src_hbm, dst_hbm,
                     src_v, dst_v, a_v, c_v, s_sh, sem1, sem2)
    plsc.subcore_barrier()
    _rows_copy(sid, [
        (lambda r: s_sh.at[pl.ds(r, CPR)],
         lambda r: s_out.at[cid, pl.ds(r, CPR)]),
    ])
    plsc.subcore_barrier()
    _rows_copy(sid, [
        (lambda r: zs_hbm.at[pl.ds(r, CPR)], lambda r: s_sh.at[pl.ds(r, CPR)]),
    ])

    def ones_row(i, cc):
        for g in range(HID // LANES):
            c_v[i, pl.ds(g * LANES, LANES)] = jnp.full((LANES,), 1.0, F32)
        return cc

    lax.fori_loop(0, CHUNK, ones_row, 0)
    plsc.subcore_barrier()

    def cnt_chunk(j, carry):
        off = wid * EPW + j * CHUNK
        pltpu.sync_copy(dst_hbm.at[pl.ds(off, CHUNK)], dst_v)
        pltpu.sync_copy(c_v, s_sh.at[dst_v], add=True)
        return carry

    lax.fori_loop(0, NCHUNK, cnt_chunk, 0)
    plsc.subcore_barrier()
    _rows_copy(sid, [
        (lambda r: s_sh.at[pl.ds(r, CPR)],
         lambda r: cnt_out.at[cid, pl.ds(r, CPR)]),
    ])


@functools.partial(
    pl.kernel,
    out_type=jax.ShapeDtypeStruct((NC, N_NODES, HID), F32),
    mesh=_SC_MESH,
    scratch_types=[
        pltpu.VMEM((CHUNK,), jnp.int32),
        pltpu.VMEM((CHUNK,), jnp.int32),
        pltpu.VMEM((CHUNK, HID), F32),
        pltpu.VMEM((CHUNK, HID), F32),
        pltpu.VMEM_SHARED((N_NODES, HID), F32),
        pltpu.SemaphoreType.DMA,
        pltpu.SemaphoreType.DMA,
    ],
)
def _sc_edge(a_hbm, c_hbm, src_hbm, dst_hbm, zs_hbm,
             s_out,
             src_v, dst_v, a_v, c_v, s_sh, sem1, sem2):
    cid = lax.axis_index("c")
    sid = lax.axis_index("s")
    wid = sid * NC + cid
    _rows_copy(sid, [
        (lambda r: zs_hbm.at[pl.ds(r, CPR)], lambda r: s_sh.at[pl.ds(r, CPR)]),
    ])
    plsc.subcore_barrier()
    _edge_chunk_loop(wid, a_hbm, c_hbm, src_hbm, dst_hbm,
                     src_v, dst_v, a_v, c_v, s_sh, sem1, sem2)
    plsc.subcore_barrier()
    _rows_copy(sid, [
        (lambda r: s_sh.at[pl.ds(r, CPR)],
         lambda r: s_out.at[cid, pl.ds(r, CPR)]),
    ])


def _dot(a, b):
    return jnp.dot(a, b, preferred_element_type=F32)


def _pre_body(x, b2d, nwt, nb, dfp, dwt, db, wat, b1, wbt,
              h_out, a_out, c_out):
    demb = _dot(dfp[...], dwt[...]) + db[...]
    h = _dot(x[...], nwt[...]) + nb[...]
    oneh = (b2d[...] == lax.broadcasted_iota(jnp.int32, (BLK, N_BATCH), 1)
            ).astype(F32)
    h = h + _dot(oneh, demb)
    h_out[...] = h
    a_out[...] = _dot(h, wat[...]) + b1[...]
    c_out[...] = _dot(h, wbt[...])


def _layer_update(h, s2, c2, w2t, b2, swt, sb, owat, owbt, ob, g, bb):
    sarr = s2[...]
    s = sarr[0] + sarr[1]
    carr = c2[...]
    cnt = carr[0, :, 0:1] + carr[1, :, 0:1]
    aggr = (_dot(s, w2t[...]) + cnt * b2[...]) / jnp.maximum(cnt, 1.0)
    selfp = _dot(h[...], swt[...]) + sb[...]
    cc = _dot(selfp, owat[...]) + _dot(aggr, owbt[...]) + ob[...]
    mu = jnp.mean(cc, axis=-1, keepdims=True)
    var = jnp.mean((cc - mu) ** 2, axis=-1, keepdims=True)
    ln = (cc - mu) * lax.rsqrt(var + 1e-5) * g[...] + bb[...]
    return h[...] + jnp.maximum(ln, 0.0)


def _mid_body(h, s2, c2, w2t, b2, swt, sb, owat, owbt, ob, g, bb,
              want, b1n, wbnt, h_out, a_out, c_out):
    hn = _layer_update(h, s2, c2, w2t, b2, swt, sb, owat, owbt, ob, g, bb)
    h_out[...] = hn
    a_out[...] = _dot(hn, want[...]) + b1n[...]
    c_out[...] = _dot(hn, wbnt[...])


def _post_body(h, s2, c2, w2t, b2, swt, sb, owat, owbt, ob, g, bb,
               pjt, pjb, y_out):
    hn = _layer_update(h, s2, c2, w2t, b2, swt, sb, owat, owbt, ob, g, bb)
    y_out[...] = _dot(hn, pjt[...]) + pjb[...]


def _rows_spec(ncol):
    return pl.BlockSpec((BLK, ncol), lambda i: (i, 0))


def _full_spec(shape):
    nd = len(shape)
    return pl.BlockSpec(shape, lambda i, _nd=nd: (0,) * _nd)


def _part_spec(ncol):
    return pl.BlockSpec((NC, BLK, ncol), lambda i: (0, i, 0))


def kernel(x, edge_index, drone_feat, batch, node_w, node_b, drone_w, drone_b,
           edge_w1, edge_b1, edge_w2, edge_b2, self_w, self_b, out_w, out_b,
           ln_g, ln_b, proj_w, proj_b):
    src = edge_index[0]
    dst = edge_index[1]
    b2d = batch.reshape(N_NODES, 1)

    nwt = node_w.T
    nb = node_b.reshape(1, HID)
    dfp = jnp.pad(drone_feat, ((0, 0), (0, N_BATCH - D_DRONE)))
    dwt = jnp.pad(drone_w.T, ((0, N_BATCH - D_DRONE), (0, 0)))
    db = drone_b.reshape(1, HID)

    wat = [(edge_w1[i, :, :HID] + edge_w1[i, :, HID:]).T for i in range(N_LAYERS)]
    wbt = [edge_w1[i, :, HID:].T for i in range(N_LAYERS)]
    b1 = [edge_b1[i].reshape(1, HID) for i in range(N_LAYERS)]
    w2t = [edge_w2[i].T for i in range(N_LAYERS)]
    b2 = [edge_b2[i].reshape(1, HID) for i in range(N_LAYERS)]
    swt = [self_w[i].T for i in range(N_LAYERS)]
    sb = [self_b[i].reshape(1, HID) for i in range(N_LAYERS)]
    owat = [out_w[i, :, :HID].T for i in range(N_LAYERS)]
    owbt = [out_w[i, :, HID:].T for i in range(N_LAYERS)]
    ob = [out_b[i].reshape(1, HID) for i in range(N_LAYERS)]
    gs = [ln_g[i].reshape(1, HID) for i in range(N_LAYERS)]
    bbs = [ln_b[i].reshape(1, HID) for i in range(N_LAYERS)]
    pjt = proj_w.T
    pjb = proj_b.reshape(1, N_OUT)

    zs = jnp.zeros((N_NODES, HID), F32)

    nrow_shape = jax.ShapeDtypeStruct((N_NODES, HID), F32)
    h, a, c = pl.pallas_call(
        _pre_body,
        grid=(GRID,),
        in_specs=[
            _rows_spec(HID), pl.BlockSpec((BLK, 1), lambda i: (i, 0)),
            _full_spec((HID, HID)), _full_spec((1, HID)),
            _full_spec((N_BATCH, N_BATCH)), _full_spec((N_BATCH, HID)),
            _full_spec((1, HID)),
            _full_spec((HID, HID)), _full_spec((1, HID)),
            _full_spec((HID, HID)),
        ],
        out_specs=[_rows_spec(HID)] * 3,
        out_shape=[nrow_shape] * 3,
    )(x, b2d, nwt, nb, dfp, dwt, db, wat[0], b1[0], wbt[0])

    s2, cnt_full = _sc_edge_cnt(a, c, src, dst, zs)
    c2 = cnt_full[:, :, :8]

    mid_in_specs = [
        _rows_spec(HID), _part_spec(HID), _part_spec(8),
        _full_spec((HID, HID)), _full_spec((1, HID)),
        _full_spec((HID, HID)), _full_spec((1, HID)),
        _full_spec((HID, HID)), _full_spec((HID, HID)), _full_spec((1, HID)),
        _full_spec((1, HID)), _full_spec((1, HID)),
    ]

    for i in range(N_LAYERS - 1):
        h, a, c = pl.pallas_call(
            _mid_body,
            grid=(GRID,),
            in_specs=mid_in_specs + [
                _full_spec((HID, HID)), _full_spec((1, HID)),
                _full_spec((HID, HID)),
            ],
            out_specs=[_rows_spec(HID)] * 3,
            out_shape=[nrow_shape] * 3,
        )(h, s2, c2, w2t[i], b2[i], swt[i], sb[i], owat[i], owbt[i], ob[i],
          gs[i], bbs[i], wat[i + 1], b1[i + 1], wbt[i + 1])
        s2 = _sc_edge(a, c, src, dst, zs)

    y = pl.pallas_call(
        _post_body,
        grid=(GRID,),
        in_specs=mid_in_specs + [
            _full_spec((HID, N_OUT)), _full_spec((1, N_OUT)),
        ],
        out_specs=_rows_spec(N_OUT),
        out_shape=jax.ShapeDtypeStruct((N_NODES, N_OUT), F32),
    )(h, s2, c2, w2t[2], b2[2], swt[2], sb[2], owat[2], owbt[2], ob[2],
      gs[2], bbs[2], pjt, pjb)
    return y


# speedup vs baseline: 6.8165x; 1.4577x over previous
import functools

import jax
import jax.numpy as jnp
from jax import lax
from jax.experimental import pallas as pl
from jax.experimental.pallas import tpu as pltpu
from jax.experimental.pallas import tpu_sc as plsc

N_NODES = 10000
N_EDGES = 320000
HID = 128
N_OUT = 64
N_BATCH = 64
D_DRONE = 51
N_LAYERS = 3
LANES = 16

NC = 2
NS = 16
NW = NC * NS
EPW = N_EDGES // NW
CHUNK = 80
NCHUNK = EPW // CHUNK
CPR = 400
NCP = N_NODES // CPR

BLK = 400
GRID = N_NODES // BLK
F32 = jnp.float32

_SC_MESH = plsc.VectorSubcoreMesh(core_axis_name="c", subcore_axis_name="s")


def _edge_chunk_loop(wid, a_hbm, c_hbm, src_hbm, dst_hbm,
                     src2, dst2, a2, c2, s_sh, gsem, ssem):
    def load_idx(t, p):
        off = wid * EPW + t * CHUNK
        pltpu.sync_copy(src_hbm.at[pl.ds(off, CHUNK)], src2[p])
        pltpu.sync_copy(dst_hbm.at[pl.ds(off, CHUNK)], dst2[p])

    def issue_gathers(p):
        pltpu.async_copy(a_hbm.at[src2[p]], a2[p], gsem[p])
        pltpu.async_copy(c_hbm.at[dst2[p]], c2[p], gsem[p])

    def wait_gathers(p):
        pltpu.make_async_copy(a_hbm.at[src2[p]], a2[p], gsem[p]).wait()
        pltpu.make_async_copy(c_hbm.at[dst2[p]], c2[p], gsem[p]).wait()

    def wait_scatter(p):
        pltpu.make_async_copy(a2[p], s_sh.at[dst2[p]], ssem[p]).wait()

    def compute(p):
        ap = a2[p]
        cp = c2[p]

        def row_body(i, cc):
            for g in range(HID // LANES):
                sl = pl.ds(g * LANES, LANES)
                ap[i, sl] = jnp.maximum(ap[i, sl] - cp[i, sl], 0.0)
            return cc

        lax.fori_loop(0, CHUNK, row_body, 0)

    load_idx(0, 0)
    issue_gathers(0)

    def outer(j4, carry):
        for b in range(4):
            p = b % 2
            q = 1 - p
            t = j4 * 4 + b

            @pl.when(t >= 1)
            def _():
                wait_scatter(q)

            load_idx(t + 1, q)
            issue_gathers(q)
            wait_gathers(p)
            compute(p)
            pltpu.async_copy(a2[p], s_sh.at[dst2[p]], ssem[p], add=True)
        return carry

    lax.fori_loop(0, (NCHUNK - 1) // 4, outer, 0)
    wait_scatter(1)
    wait_gathers(0)
    compute(0)
    pltpu.sync_copy(a2[0], s_sh.at[dst2[0]], add=True)


def _rows_copy(sid, pairs):
    r0 = sid * CPR
    for src, dst in pairs:
        pltpu.sync_copy(src(r0), dst(r0))

    @pl.when(sid + NS < NCP)
    def _():
        r1 = (sid + NS) * CPR
        for src, dst in pairs:
            pltpu.sync_copy(src(r1), dst(r1))


@functools.partial(
    pl.kernel,
    out_type=(
        jax.ShapeDtypeStruct((NC, N_NODES, HID), F32),
        jax.ShapeDtypeStruct((NC, N_NODES, HID), F32),
    ),
    mesh=_SC_MESH,
    scratch_types=[
        pltpu.VMEM((CHUNK,), jnp.int32),
        pltpu.VMEM((CHUNK,), jnp.int32),
        pltpu.VMEM((CHUNK,), jnp.int32),
        pltpu.VMEM((CHUNK,), jnp.int32),
        pltpu.VMEM((CHUNK, HID), F32),
        pltpu.VMEM((CHUNK, HID), F32),
        pltpu.VMEM((CHUNK, HID), F32),
        pltpu.VMEM((CHUNK, HID), F32),
        pltpu.VMEM_SHARED((N_NODES, HID), F32),
        pltpu.SemaphoreType.DMA,
        pltpu.SemaphoreType.DMA,
        pltpu.SemaphoreType.DMA,
        pltpu.SemaphoreType.DMA,
    ],
)
def _sc_edge_cnt(a_hbm, c_hbm, src_hbm, dst_hbm, zs_hbm,
                 s_out, cnt_out,
                 src_a, src_b, dst_a, dst_b, a_a, a_b, c_a, c_b,
                 s_sh, gsem_a, gsem_b, ssem_a, ssem_b):
    cid = lax.axis_index("c")
    sid = lax.axis_index("s")
    wid = sid * NC + cid
    _rows_copy(sid, [
        (lambda r: zs_hbm.at[pl.ds(r, CPR)], lambda r: s_sh.at[pl.ds(r, CPR)]),
    ])
    plsc.subcore_barrier()
    _edge_chunk_loop(wid, a_hbm, c_hbm, src_hbm, dst_hbm,
                     (src_a, src_b), (dst_a, dst_b), (a_a, a_b), (c_a, c_b),
                     s_sh, (gsem_a, gsem_b), (ssem_a, ssem_b))
    plsc.subcore_barrier()
    _rows_copy(sid, [
        (lambda r: s_sh.at[pl.ds(r, CPR)],
         lambda r: s_out.at[cid, pl.ds(r, CPR)]),
    ])
    plsc.subcore_barrier()
    _rows_copy(sid, [
        (lambda r: zs_hbm.at[pl.ds(r, CPR)], lambda r: s_sh.at[pl.ds(r, CPR)]),
    ])

    def ones_row(i, cc):
        for g in range(HID // LANES):
            c_a[i, pl.ds(g * LANES, LANES)] = jnp.full((LANES,), 1.0, F32)
        return cc

    lax.fori_loop(0, CHUNK, ones_row, 0)
    plsc.subcore_barrier()

    def cnt_chunk(j, carry):
        off = wid * EPW + j * CHUNK
        pltpu.sync_copy(dst_hbm.at[pl.ds(off, CHUNK)], dst_a)
        pltpu.sync_copy(c_a, s_sh.at[dst_a], add=True)
        return carry

    lax.fori_loop(0, NCHUNK, cnt_chunk, 0)
    plsc.subcore_barrier()
    _rows_copy(sid, [
        (lambda r: s_sh.at[pl.ds(r, CPR)],
         lambda r: cnt_out.at[cid, pl.ds(r, CPR)]),
    ])


@functools.partial(
    pl.kernel,
    out_type=jax.ShapeDtypeStruct((NC, N_NODES, HID), F32),
    mesh=_SC_MESH,
    scratch_types=[
        pltpu.VMEM((CHUNK,), jnp.int32),
        pltpu.VMEM((CHUNK,), jnp.int32),
        pltpu.VMEM((CHUNK,), jnp.int32),
        pltpu.VMEM((CHUNK,), jnp.int32),
        pltpu.VMEM((CHUNK, HID), F32),
        pltpu.VMEM((CHUNK, HID), F32),
        pltpu.VMEM((CHUNK, HID), F32),
        pltpu.VMEM((CHUNK, HID), F32),
        pltpu.VMEM_SHARED((N_NODES, HID), F32),
        pltpu.SemaphoreType.DMA,
        pltpu.SemaphoreType.DMA,
        pltpu.SemaphoreType.DMA,
        pltpu.SemaphoreType.DMA,
    ],
)
def _sc_edge(a_hbm, c_hbm, src_hbm, dst_hbm, zs_hbm,
             s_out,
             src_a, src_b, dst_a, dst_b, a_a, a_b, c_a, c_b,
             s_sh, gsem_a, gsem_b, ssem_a, ssem_b):
    cid = lax.axis_index("c")
    sid = lax.axis_index("s")
    wid = sid * NC + cid
    _rows_copy(sid, [
        (lambda r: zs_hbm.at[pl.ds(r, CPR)], lambda r: s_sh.at[pl.ds(r, CPR)]),
    ])
    plsc.subcore_barrier()
    _edge_chunk_loop(wid, a_hbm, c_hbm, src_hbm, dst_hbm,
                     (src_a, src_b), (dst_a, dst_b), (a_a, a_b), (c_a, c_b),
                     s_sh, (gsem_a, gsem_b), (ssem_a, ssem_b))
    plsc.subcore_barrier()
    _rows_copy(sid, [
        (lambda r: s_sh.at[pl.ds(r, CPR)],
         lambda r: s_out.at[cid, pl.ds(r, CPR)]),
    ])


def _dot(a, b):
    return jnp.dot(a, b, preferred_element_type=F32)


def _pre_body(x, b2d, nwt, nb, dfp, dwt, db, wat, b1, wbt,
              h_out, a_out, c_out):
    demb = _dot(dfp[...], dwt[...]) + db[...]
    h = _dot(x[...], nwt[...]) + nb[...]
    oneh = (b2d[...] == lax.broadcasted_iota(jnp.int32, (BLK, N_BATCH), 1)
            ).astype(F32)
    h = h + _dot(oneh, demb)
    h_out[...] = h
    a_out[...] = _dot(h, wat[...]) + b1[...]
    c_out[...] = _dot(h, wbt[...])


def _layer_update(h, s2, c2, w2t, b2, swt, sb, owat, owbt, ob, g, bb):
    sarr = s2[...]
    s = sarr[0] + sarr[1]
    carr = c2[...]
    cnt = carr[0, :, 0:1] + carr[1, :, 0:1]
    aggr = (_dot(s, w2t[...]) + cnt * b2[...]) / jnp.maximum(cnt, 1.0)
    selfp = _dot(h[...], swt[...]) + sb[...]
    cc = _dot(selfp, owat[...]) + _dot(aggr, owbt[...]) + ob[...]
    mu = jnp.mean(cc, axis=-1, keepdims=True)
    var = jnp.mean((cc - mu) ** 2, axis=-1, keepdims=True)
    ln = (cc - mu) * lax.rsqrt(var + 1e-5) * g[...] + bb[...]
    return h[...] + jnp.maximum(ln, 0.0)


def _mid_body(h, s2, c2, w2t, b2, swt, sb, owat, owbt, ob, g, bb,
              want, b1n, wbnt, h_out, a_out, c_out):
    hn = _layer_update(h, s2, c2, w2t, b2, swt, sb, owat, owbt, ob, g, bb)
    h_out[...] = hn
    a_out[...] = _dot(hn, want[...]) + b1n[...]
    c_out[...] = _dot(hn, wbnt[...])


def _post_body(h, s2, c2, w2t, b2, swt, sb, owat, owbt, ob, g, bb,
               pjt, pjb, y_out):
    hn = _layer_update(h, s2, c2, w2t, b2, swt, sb, owat, owbt, ob, g, bb)
    y_out[...] = _dot(hn, pjt[...]) + pjb[...]


def _rows_spec(ncol):
    return pl.BlockSpec((BLK, ncol), lambda i: (i, 0))


def _full_spec(shape):
    nd = len(shape)
    return pl.BlockSpec(shape, lambda i, _nd=nd: (0,) * _nd)


def _part_spec(ncol):
    return pl.BlockSpec((NC, BLK, ncol), lambda i: (0, i, 0))


def kernel(x, edge_index, drone_feat, batch, node_w, node_b, drone_w, drone_b,
           edge_w1, edge_b1, edge_w2, edge_b2, self_w, self_b, out_w, out_b,
           ln_g, ln_b, proj_w, proj_b):
    src = edge_index[0]
    dst = edge_index[1]
    b2d = batch.reshape(N_NODES, 1)

    nwt = node_w.T
    nb = node_b.reshape(1, HID)
    dfp = jnp.pad(drone_feat, ((0, 0), (0, N_BATCH - D_DRONE)))
    dwt = jnp.pad(drone_w.T, ((0, N_BATCH - D_DRONE), (0, 0)))
    db = drone_b.reshape(1, HID)

    wat = [(edge_w1[i, :, :HID] + edge_w1[i, :, HID:]).T for i in range(N_LAYERS)]
    wbt = [edge_w1[i, :, HID:].T for i in range(N_LAYERS)]
    b1 = [edge_b1[i].reshape(1, HID) for i in range(N_LAYERS)]
    w2t = [edge_w2[i].T for i in range(N_LAYERS)]
    b2 = [edge_b2[i].reshape(1, HID) for i in range(N_LAYERS)]
    swt = [self_w[i].T for i in range(N_LAYERS)]
    sb = [self_b[i].reshape(1, HID) for i in range(N_LAYERS)]
    owat = [out_w[i, :, :HID].T for i in range(N_LAYERS)]
    owbt = [out_w[i, :, HID:].T for i in range(N_LAYERS)]
    ob = [out_b[i].reshape(1, HID) for i in range(N_LAYERS)]
    gs = [ln_g[i].reshape(1, HID) for i in range(N_LAYERS)]
    bbs = [ln_b[i].reshape(1, HID) for i in range(N_LAYERS)]
    pjt = proj_w.T
    pjb = proj_b.reshape(1, N_OUT)

    zs = jnp.zeros((N_NODES, HID), F32)

    nrow_shape = jax.ShapeDtypeStruct((N_NODES, HID), F32)
    h, a, c = pl.pallas_call(
        _pre_body,
        grid=(GRID,),
        in_specs=[
            _rows_spec(HID), pl.BlockSpec((BLK, 1), lambda i: (i, 0)),
            _full_spec((HID, HID)), _full_spec((1, HID)),
            _full_spec((N_BATCH, N_BATCH)), _full_spec((N_BATCH, HID)),
            _full_spec((1, HID)),
            _full_spec((HID, HID)), _full_spec((1, HID)),
            _full_spec((HID, HID)),
        ],
        out_specs=[_rows_spec(HID)] * 3,
        out_shape=[nrow_shape] * 3,
    )(x, b2d, nwt, nb, dfp, dwt, db, wat[0], b1[0], wbt[0])

    s2, cnt_full = _sc_edge_cnt(a, c, src, dst, zs)
    c2 = cnt_full[:, :, :8]

    mid_in_specs = [
        _rows_spec(HID), _part_spec(HID), _part_spec(8),
        _full_spec((HID, HID)), _full_spec((1, HID)),
        _full_spec((HID, HID)), _full_spec((1, HID)),
        _full_spec((HID, HID)), _full_spec((HID, HID)), _full_spec((1, HID)),
        _full_spec((1, HID)), _full_spec((1, HID)),
    ]

    for i in range(N_LAYERS - 1):
        h, a, c = pl.pallas_call(
            _mid_body,
            grid=(GRID,),
            in_specs=mid_in_specs + [
                _full_spec((HID, HID)), _full_spec((1, HID)),
                _full_spec((HID, HID)),
            ],
            out_specs=[_rows_spec(HID)] * 3,
            out_shape=[nrow_shape] * 3,
        )(h, s2, c2, w2t[i], b2[i], swt[i], sb[i], owat[i], owbt[i], ob[i],
          gs[i], bbs[i], wat[i + 1], b1[i + 1], wbt[i + 1])
        s2 = _sc_edge(a, c, src, dst, zs)

    y = pl.pallas_call(
        _post_body,
        grid=(GRID,),
        in_specs=mid_in_specs + [
            _full_spec((HID, N_OUT)), _full_spec((1, N_OUT)),
        ],
        out_specs=_rows_spec(N_OUT),
        out_shape=jax.ShapeDtypeStruct((N_NODES, N_OUT), F32),
    )(h, s2, c2, w2t[2], b2[2], swt[2], sb[2], owat[2], owbt[2], ob[2],
      gs[2], bbs[2], pjt, pjb)
    return y


# speedup vs baseline: 7.1493x; 1.0488x over previous
import functools

import jax
import jax.numpy as jnp
from jax import lax
from jax.experimental import pallas as pl
from jax.experimental.pallas import tpu as pltpu
from jax.experimental.pallas import tpu_sc as plsc

N_NODES = 10000
N_EDGES = 320000
HID = 128
N_OUT = 64
N_BATCH = 64
D_DRONE = 51
N_LAYERS = 3
LANES = 16

NC = 2
NS = 16
NW = NC * NS
EPW = N_EDGES // NW
CHUNK = 80
NCHUNK = EPW // CHUNK
CPR = 400
NCP = N_NODES // CPR

BLK = 400
GRID = N_NODES // BLK
F32 = jnp.float32

_SC_MESH = plsc.VectorSubcoreMesh(core_axis_name="c", subcore_axis_name="s")


NBUF = 2
assert (NCHUNK - (NBUF - 1)) % NBUF == 0


def _edge_chunk_loop(wid, a_hbm, c_hbm, src_hbm, dst_hbm,
                     srcs, dsts, avs, cvs, s_sh, gsems, ssems):
    def load_idx(t, p):
        off = wid * EPW + t * CHUNK
        pltpu.sync_copy(src_hbm.at[pl.ds(off, CHUNK)], srcs[p])
        pltpu.sync_copy(dst_hbm.at[pl.ds(off, CHUNK)], dsts[p])

    def issue_gathers(p):
        pltpu.async_copy(a_hbm.at[srcs[p]], avs[p], gsems[p])
        pltpu.async_copy(c_hbm.at[dsts[p]], cvs[p], gsems[p])

    def wait_gathers(p):
        pltpu.make_async_copy(a_hbm.at[srcs[p]], avs[p], gsems[p]).wait()
        pltpu.make_async_copy(c_hbm.at[dsts[p]], cvs[p], gsems[p]).wait()

    def issue_scatter(p):
        pltpu.async_copy(avs[p], s_sh.at[dsts[p]], ssems[p], add=True)

    def wait_scatter(p):
        pltpu.make_async_copy(avs[p], s_sh.at[dsts[p]], ssems[p]).wait()

    def compute(p):
        ap = avs[p]
        cp = cvs[p]

        def row_body(i, cc):
            for g in range(HID // LANES):
                sl = pl.ds(g * LANES, LANES)
                ap[i, sl] = jnp.maximum(ap[i, sl] - cp[i, sl], 0.0)
            return cc

        lax.fori_loop(0, CHUNK, row_body, 0)

    for t in range(NBUF - 1):
        load_idx(t, t)
        issue_gathers(t)

    def outer(k, carry):
        for b in range(NBUF):
            t = k * NBUF + b
            p = b
            r = (b + NBUF - 1) % NBUF

            @pl.when(t >= 1)
            def _():
                wait_scatter(r)

            load_idx(t + NBUF - 1, r)
            issue_gathers(r)
            wait_gathers(p)
            compute(p)
            issue_scatter(p)
        return carry

    nbody = NCHUNK - (NBUF - 1)
    lax.fori_loop(0, nbody // NBUF, outer, 0)
    for t in range(nbody, NCHUNK):
        p = t % NBUF
        wait_gathers(p)
        compute(p)
        issue_scatter(p)
    for p in range(NBUF):
        wait_scatter(p)


def _rows_copy(sid, pairs):
    r0 = sid * CPR
    for src, dst in pairs:
        pltpu.sync_copy(src(r0), dst(r0))

    @pl.when(sid + NS < NCP)
    def _():
        r1 = (sid + NS) * CPR
        for src, dst in pairs:
            pltpu.sync_copy(src(r1), dst(r1))


_SC_SCRATCH = (
    [pltpu.VMEM((CHUNK,), jnp.int32) for _ in range(2 * NBUF)]
    + [pltpu.VMEM((CHUNK, HID), F32) for _ in range(2 * NBUF)]
    + [pltpu.VMEM_SHARED((N_NODES, HID), F32)]
    + [pltpu.SemaphoreType.DMA for _ in range(2 * NBUF)]
)


@functools.partial(
    pl.kernel,
    out_type=(
        jax.ShapeDtypeStruct((NC, N_NODES, HID), F32),
        jax.ShapeDtypeStruct((NC, N_NODES, HID), F32),
    ),
    mesh=_SC_MESH,
    scratch_types=_SC_SCRATCH,
)
def _sc_edge_cnt(a_hbm, c_hbm, src_hbm, dst_hbm, zs_hbm,
                 s_out, cnt_out, *bufs):
    srcs, dsts = bufs[0:NBUF], bufs[NBUF:2 * NBUF]
    avs, cvs = bufs[2 * NBUF:3 * NBUF], bufs[3 * NBUF:4 * NBUF]
    s_sh = bufs[4 * NBUF]
    gsems = bufs[4 * NBUF + 1:5 * NBUF + 1]
    ssems = bufs[5 * NBUF + 1:6 * NBUF + 1]
    cid = lax.axis_index("c")
    sid = lax.axis_index("s")
    wid = sid * NC + cid
    _rows_copy(sid, [
        (lambda r: zs_hbm.at[pl.ds(r, CPR)], lambda r: s_sh.at[pl.ds(r, CPR)]),
    ])
    plsc.subcore_barrier()
    _edge_chunk_loop(wid, a_hbm, c_hbm, src_hbm, dst_hbm,
                     srcs, dsts, avs, cvs, s_sh, gsems, ssems)
    plsc.subcore_barrier()
    _rows_copy(sid, [
        (lambda r: s_sh.at[pl.ds(r, CPR)],
         lambda r: s_out.at[cid, pl.ds(r, CPR)]),
    ])
    plsc.subcore_barrier()
    _rows_copy(sid, [
        (lambda r: zs_hbm.at[pl.ds(r, CPR)], lambda r: s_sh.at[pl.ds(r, CPR)]),
    ])

    ones_v = cvs[0]

    def ones_row(i, cc):
        for g in range(HID // LANES):
            ones_v[i, pl.ds(g * LANES, LANES)] = jnp.full((LANES,), 1.0, F32)
        return cc

    lax.fori_loop(0, CHUNK, ones_row, 0)
    plsc.subcore_barrier()

    def cnt_load_issue(t, p):
        off = wid * EPW + t * CHUNK
        pltpu.sync_copy(dst_hbm.at[pl.ds(off, CHUNK)], dsts[p])
        pltpu.async_copy(ones_v, s_sh.at[dsts[p]], ssems[p], add=True)

    def cnt_wait(p):
        pltpu.make_async_copy(ones_v, s_sh.at[dsts[p]], ssems[p]).wait()

    def cnt_outer(k, carry):
        for b in range(2):
            t = k * 2 + b

            @pl.when(t >= 2)
            def _():
                cnt_wait(b)

            cnt_load_issue(t, b)
        return carry

    lax.fori_loop(0, (NCHUNK - 1) // 2, cnt_outer, 0)
    cnt_wait(0)
    cnt_load_issue(NCHUNK - 1, 0)
    cnt_wait(0)
    cnt_wait(1)
    plsc.subcore_barrier()
    _rows_copy(sid, [
        (lambda r: s_sh.at[pl.ds(r, CPR)],
         lambda r: cnt_out.at[cid, pl.ds(r, CPR)]),
    ])


@functools.partial(
    pl.kernel,
    out_type=jax.ShapeDtypeStruct((NC, N_NODES, HID), F32),
    mesh=_SC_MESH,
    scratch_types=_SC_SCRATCH,
)
def _sc_edge(a_hbm, c_hbm, src_hbm, dst_hbm, zs_hbm,
             s_out, *bufs):
    srcs, dsts = bufs[0:NBUF], bufs[NBUF:2 * NBUF]
    avs, cvs = bufs[2 * NBUF:3 * NBUF], bufs[3 * NBUF:4 * NBUF]
    s_sh = bufs[4 * NBUF]
    gsems = bufs[4 * NBUF + 1:5 * NBUF + 1]
    ssems = bufs[5 * NBUF + 1:6 * NBUF + 1]
    cid = lax.axis_index("c")
    sid = lax.axis_index("s")
    wid = sid * NC + cid
    _rows_copy(sid, [
        (lambda r: zs_hbm.at[pl.ds(r, CPR)], lambda r: s_sh.at[pl.ds(r, CPR)]),
    ])
    plsc.subcore_barrier()
    _edge_chunk_loop(wid, a_hbm, c_hbm, src_hbm, dst_hbm,
                     srcs, dsts, avs, cvs, s_sh, gsems, ssems)
    plsc.subcore_barrier()
    _rows_copy(sid, [
        (lambda r: s_sh.at[pl.ds(r, CPR)],
         lambda r: s_out.at[cid, pl.ds(r, CPR)]),
    ])


def _dot(a, b):
    return jnp.dot(a, b, preferred_element_type=F32)


def _pre_body(x, b2d, nwt, nb, dfp, dwt, db, wat, b1, wbt,
              h_out, a_out, c_out):
    demb = _dot(dfp[...], dwt[...]) + db[...]
    h = _dot(x[...], nwt[...]) + nb[...]
    oneh = (b2d[...] == lax.broadcasted_iota(jnp.int32, (BLK, N_BATCH), 1)
            ).astype(F32)
    h = h + _dot(oneh, demb)
    h_out[...] = h
    a_out[...] = _dot(h, wat[...]) + b1[...]
    c_out[...] = _dot(h, wbt[...])


def _layer_update(h, s2, c2, w2t, b2, swt, sb, owat, owbt, ob, g, bb):
    sarr = s2[...]
    s = sarr[0] + sarr[1]
    carr = c2[...]
    cnt = carr[0, :, 0:1] + carr[1, :, 0:1]
    aggr = (_dot(s, w2t[...]) + cnt * b2[...]) / jnp.maximum(cnt, 1.0)
    selfp = _dot(h[...], swt[...]) + sb[...]
    cc = _dot(selfp, owat[...]) + _dot(aggr, owbt[...]) + ob[...]
    mu = jnp.mean(cc, axis=-1, keepdims=True)
    var = jnp.mean((cc - mu) ** 2, axis=-1, keepdims=True)
    ln = (cc - mu) * lax.rsqrt(var + 1e-5) * g[...] + bb[...]
    return h[...] + jnp.maximum(ln, 0.0)


def _mid_body(h, s2, c2, w2t, b2, swt, sb, owat, owbt, ob, g, bb,
              want, b1n, wbnt, h_out, a_out, c_out):
    hn = _layer_update(h, s2, c2, w2t, b2, swt, sb, owat, owbt, ob, g, bb)
    h_out[...] = hn
    a_out[...] = _dot(hn, want[...]) + b1n[...]
    c_out[...] = _dot(hn, wbnt[...])


def _post_body(h, s2, c2, w2t, b2, swt, sb, owat, owbt, ob, g, bb,
               pjt, pjb, y_out):
    hn = _layer_update(h, s2, c2, w2t, b2, swt, sb, owat, owbt, ob, g, bb)
    y_out[...] = _dot(hn, pjt[...]) + pjb[...]


def _rows_spec(ncol):
    return pl.BlockSpec((BLK, ncol), lambda i: (i, 0))


def _full_spec(shape):
    nd = len(shape)
    return pl.BlockSpec(shape, lambda i, _nd=nd: (0,) * _nd)


def _part_spec(ncol):
    return pl.BlockSpec((NC, BLK, ncol), lambda i: (0, i, 0))


def kernel(x, edge_index, drone_feat, batch, node_w, node_b, drone_w, drone_b,
           edge_w1, edge_b1, edge_w2, edge_b2, self_w, self_b, out_w, out_b,
           ln_g, ln_b, proj_w, proj_b):
    src = edge_index[0]
    dst = edge_index[1]
    b2d = batch.reshape(N_NODES, 1)

    nwt = node_w.T
    nb = node_b.reshape(1, HID)
    dfp = jnp.pad(drone_feat, ((0, 0), (0, N_BATCH - D_DRONE)))
    dwt = jnp.pad(drone_w.T, ((0, N_BATCH - D_DRONE), (0, 0)))
    db = drone_b.reshape(1, HID)

    wat = [(edge_w1[i, :, :HID] + edge_w1[i, :, HID:]).T for i in range(N_LAYERS)]
    wbt = [edge_w1[i, :, HID:].T for i in range(N_LAYERS)]
    b1 = [edge_b1[i].reshape(1, HID) for i in range(N_LAYERS)]
    w2t = [edge_w2[i].T for i in range(N_LAYERS)]
    b2 = [edge_b2[i].reshape(1, HID) for i in range(N_LAYERS)]
    swt = [self_w[i].T for i in range(N_LAYERS)]
    sb = [self_b[i].reshape(1, HID) for i in range(N_LAYERS)]
    owat = [out_w[i, :, :HID].T for i in range(N_LAYERS)]
    owbt = [out_w[i, :, HID:].T for i in range(N_LAYERS)]
    ob = [out_b[i].reshape(1, HID) for i in range(N_LAYERS)]
    gs = [ln_g[i].reshape(1, HID) for i in range(N_LAYERS)]
    bbs = [ln_b[i].reshape(1, HID) for i in range(N_LAYERS)]
    pjt = proj_w.T
    pjb = proj_b.reshape(1, N_OUT)

    zs = jnp.zeros((N_NODES, HID), F32)

    nrow_shape = jax.ShapeDtypeStruct((N_NODES, HID), F32)
    h, a, c = pl.pallas_call(
        _pre_body,
        grid=(GRID,),
        in_specs=[
            _rows_spec(HID), pl.BlockSpec((BLK, 1), lambda i: (i, 0)),
            _full_spec((HID, HID)), _full_spec((1, HID)),
            _full_spec((N_BATCH, N_BATCH)), _full_spec((N_BATCH, HID)),
            _full_spec((1, HID)),
            _full_spec((HID, HID)), _full_spec((1, HID)),
            _full_spec((HID, HID)),
        ],
        out_specs=[_rows_spec(HID)] * 3,
        out_shape=[nrow_shape] * 3,
    )(x, b2d, nwt, nb, dfp, dwt, db, wat[0], b1[0], wbt[0])

    s2, cnt_full = _sc_edge_cnt(a, c, src, dst, zs)
    c2 = cnt_full[:, :, :8]

    mid_in_specs = [
        _rows_spec(HID), _part_spec(HID), _part_spec(8),
        _full_spec((HID, HID)), _full_spec((1, HID)),
        _full_spec((HID, HID)), _full_spec((1, HID)),
        _full_spec((HID, HID)), _full_spec((HID, HID)), _full_spec((1, HID)),
        _full_spec((1, HID)), _full_spec((1, HID)),
    ]

    for i in range(N_LAYERS - 1):
        h, a, c = pl.pallas_call(
            _mid_body,
            grid=(GRID,),
            in_specs=mid_in_specs + [
                _full_spec((HID, HID)), _full_spec((1, HID)),
                _full_spec((HID, HID)),
            ],
            out_specs=[_rows_spec(HID)] * 3,
            out_shape=[nrow_shape] * 3,
        )(h, s2, c2, w2t[i], b2[i], swt[i], sb[i], owat[i], owbt[i], ob[i],
          gs[i], bbs[i], wat[i + 1], b1[i + 1], wbt[i + 1])
        s2 = _sc_edge(a, c, src, dst, zs)

    y = pl.pallas_call(
        _post_body,
        grid=(GRID,),
        in_specs=mid_in_specs + [
            _full_spec((HID, N_OUT)), _full_spec((1, N_OUT)),
        ],
        out_specs=_rows_spec(N_OUT),
        out_shape=jax.ShapeDtypeStruct((N_NODES, N_OUT), F32),
    )(h, s2, c2, w2t[2], b2[2], swt[2], sb[2], owat[2], owbt[2], ob[2],
      gs[2], bbs[2], pjt, pjb)
    return y
